# baseline (speedup 1.0000x reference)
import functools

import jax
import jax.numpy as jnp
from jax import lax
from jax.experimental import pallas as pl
from jax.experimental.pallas import tpu as pltpu
from jax.experimental.pallas import tpu_sc as plsc

B = 4096
L = 200
E = 32
V = 1_000_000
N = B * L
NW = 32
FULL_TILES = V // 128
TAIL = V - FULL_TILES * 128
TPW = FULL_TILES // NW + 1

_mesh = plsc.VectorSubcoreMesh(core_axis_name="c", subcore_axis_name="s")


def _iota16():
    return lax.broadcasted_iota(jnp.int32, (16,), 0)


@functools.partial(
    pl.kernel,
    mesh=_mesh,
    out_type=jax.ShapeDtypeStruct((V * E // 128, 128), jnp.float32),
    scratch_types=[
        [pltpu.VMEM((E, 128), jnp.float32) for _ in range(2)],
        [pltpu.VMEM((E, 128), jnp.float32) for _ in range(2)],
        [pltpu.SemaphoreType.DMA for _ in range(2)],
        [pltpu.SemaphoreType.DMA for _ in range(2)],
    ],
    compiler_params=pltpu.CompilerParams(
        use_tc_tiling_on_sc=True, needs_layout_passes=False
    ),
)
def _transpose_table(tokT, tail16, out, tin, tout, sem_i, sem_o):
    wid = lax.axis_index("s") * 2 + lax.axis_index("c")

    def col(t):
        return wid + NW * t

    def fire(t, bb):
        @pl.when(col(t) < FULL_TILES)
        def _():
            pltpu.async_copy(tokT.at[:, pl.ds(col(t) * 128, 128)], tin[bb], sem_i[bb])

    def process(t, bb):
        c = col(t)

        @pl.when(c < FULL_TILES)
        def _():
            pltpu.make_async_copy(
                tokT.at[:, pl.ds(c * 128, 128)], tin[bb], sem_i[bb]
            ).wait()

            def drain_prev():
                pltpu.make_async_copy(
                    tout[bb], out.at[pl.ds((c - 2 * NW) * E, E)], sem_o[bb]
                ).wait()

            if isinstance(t, int):
                if t >= 2:
                    drain_prev()
            else:
                pl.when(t >= 2)(drain_prev)

            for k in range(8):
                rows = _iota16() + 16 * (k % 2)
                for r in range(E):
                    v = plsc.load_gather(tin[bb], [rows, jnp.full((16,), 4 * r + k // 2, jnp.int32)])
                    tout[bb][r, pl.ds(16 * k, 16)] = v
            pltpu.async_copy(tout[bb], out.at[pl.ds(c * E, E)], sem_o[bb])

    fire(0, 0)

    def pair(p, carry):
        fire(2 * p + 1, 1)
        process(2 * p, 0)
        fire(2 * p + 2, 0)
        process(2 * p + 1, 1)
        return carry

    lax.fori_loop(0, TPW // 2, pair, 0)
    process(TPW - 1, 0)
    for t_last in (TPW - 2, TPW - 1):
        bb = t_last % 2

        @pl.when(col(t_last) < FULL_TILES)
        def _():
            pltpu.make_async_copy(
                tout[bb], out.at[pl.ds(col(t_last) * E, E)], sem_o[bb]
            ).wait()

    @pl.when(wid == 4)
    def _():
        nrows = TAIL * E // 128
        pltpu.sync_copy(tail16, tin[0].at[pl.ds(0, nrows)])
        pltpu.sync_copy(
            tin[0].at[pl.ds(0, nrows)], out.at[pl.ds(FULL_TILES * E, nrows)]
        )


@functools.partial(
    pl.kernel,
    mesh=_mesh,
    out_type=jax.ShapeDtypeStruct((L, E // 8, B // 128, 8, 128), jnp.float32),
    scratch_types=[
        pltpu.VMEM((L * 128,), jnp.int32),
        pltpu.VMEM((L, E), jnp.float32),
        [pltpu.VMEM((128,), jnp.int32) for _ in range(2)],
        [pltpu.VMEM((128, E), jnp.float32) for _ in range(2)],
        [pltpu.VMEM((E // 8, 8, 128), jnp.float32) for _ in range(2)],
        [pltpu.SemaphoreType.DMA for _ in range(2)],
        [pltpu.SemaphoreType.DMA for _ in range(2)],
    ],
    compiler_params=pltpu.CompilerParams(
        use_tc_tiling_on_sc=False, needs_layout_passes=False
    ),
)
def _gather_embed(x1d, tab, pos, out, xblk, pos_v, icol, rbuf, obuf, sem_g, sem_w):
    wid = lax.axis_index("s") * 2 + lax.axis_index("c")
    pltpu.sync_copy(x1d.at[pl.ds(wid * 128 * L, 128 * L)], xblk)
    pltpu.sync_copy(pos, pos_v)

    def fire(l, bb):
        for g in range(8):
            idx = (_iota16() + 16 * g) * L + l
            icol[bb][pl.ds(16 * g, 16)] = plsc.load_gather(xblk, [idx])
        pltpu.async_copy(tab.at[icol[bb]], rbuf[bb], sem_g[bb])

    def process(l, bb):
        pltpu.make_async_copy(tab.at[icol[bb]], rbuf[bb], sem_g[bb]).wait()

        @pl.when(l >= 2)
        def _():
            for a in range(E // 8):
                pltpu.make_async_copy(
                    obuf[bb].at[a], out.at[l - 2, a, wid], sem_w[bb]
                ).wait()

        l_vec = _iota16() * 0 + l
        for a in range(E // 8):
            for b8 in range(8):
                e = 8 * a + b8
                p_e = plsc.load_gather(pos_v, [l_vec, jnp.full((16,), e, jnp.int32)])
                for g in range(8):
                    rows = _iota16() + 16 * g
                    v = plsc.load_gather(rbuf[bb], [rows, jnp.full((16,), e, jnp.int32)])
                    obuf[bb][a, b8, pl.ds(16 * g, 16)] = v + p_e
        for a in range(E // 8):
            pltpu.async_copy(obuf[bb].at[a], out.at[l, a, wid], sem_w[bb])

    fire(0, 0)

    def pair(p, carry):
        fire(2 * p + 1, 1)
        process(2 * p, 0)

        @pl.when(p < L // 2 - 1)
        def _():
            fire(2 * p + 2, 0)

        process(2 * p + 1, 1)
        return carry

    lax.fori_loop(0, L // 2, pair, 0)
    for bb in range(2):
        for a in range(E // 8):
            pltpu.make_async_copy(
                obuf[bb].at[a], out.at[L - 2 + bb, a, wid], sem_w[bb]
            ).wait()


def kernel(x, token_emb, pos_emb):
    tail16 = token_emb[FULL_TILES * 128:].reshape(TAIL * E // 128, 128)
    lin = _transpose_table(token_emb.T, tail16)
    tab = lin.reshape(V, E)
    x1d = x.reshape(N).astype(jnp.int32)
    P = _gather_embed(x1d, tab, pos_emb)
    return P.transpose(2, 4, 0, 1, 3).reshape(B, L, E)


# speedup vs baseline: 1.9488x; 1.9488x over previous
import functools

import jax
import jax.numpy as jnp
from jax import lax
from jax.experimental import pallas as pl
from jax.experimental.pallas import tpu as pltpu
from jax.experimental.pallas import tpu_sc as plsc

B = 4096
L = 200
E = 32
V = 1_000_000
N = B * L
NW = 32
FULL_TILES = V // 128
TAIL = V - FULL_TILES * 128
TPW = FULL_TILES // NW + 1

_mesh = plsc.VectorSubcoreMesh(core_axis_name="c", subcore_axis_name="s")


def _iota16():
    return lax.broadcasted_iota(jnp.int32, (16,), 0)


@functools.partial(
    pl.kernel,
    mesh=_mesh,
    out_type=jax.ShapeDtypeStruct((V * E // 128, 128), jnp.float32),
    scratch_types=[
        [pltpu.VMEM((E, 128), jnp.float32) for _ in range(2)],
        [pltpu.VMEM((E, 128), jnp.float32) for _ in range(2)],
        [pltpu.SemaphoreType.DMA for _ in range(2)],
        [pltpu.SemaphoreType.DMA for _ in range(2)],
    ],
    compiler_params=pltpu.CompilerParams(
        use_tc_tiling_on_sc=True, needs_layout_passes=False
    ),
)
def _transpose_table(tokT, tail16, out, tin, tout, sem_i, sem_o):
    wid = lax.axis_index("s") * 2 + lax.axis_index("c")

    def col(t):
        return wid + NW * t

    def fire(t, bb):
        @pl.when(col(t) < FULL_TILES)
        def _():
            pltpu.async_copy(tokT.at[:, pl.ds(col(t) * 128, 128)], tin[bb], sem_i[bb])

    def process(t, bb):
        c = col(t)

        @pl.when(c < FULL_TILES)
        def _():
            pltpu.make_async_copy(
                tokT.at[:, pl.ds(c * 128, 128)], tin[bb], sem_i[bb]
            ).wait()

            def drain_prev():
                pltpu.make_async_copy(
                    tout[bb], out.at[pl.ds((c - 2 * NW) * E, E)], sem_o[bb]
                ).wait()

            if isinstance(t, int):
                if t >= 2:
                    drain_prev()
            else:
                pl.when(t >= 2)(drain_prev)

            @plsc.parallel_loop(0, E, 1, unroll=8)
            def _(r):
                for k in range(8):
                    rows = _iota16() + 16 * (k % 2)
                    cols = _iota16() * 0 + (4 * r + k // 2)
                    tout[bb][r, pl.ds(16 * k, 16)] = plsc.load_gather(
                        tin[bb], [rows, cols]
                    )

            pltpu.async_copy(tout[bb], out.at[pl.ds(c * E, E)], sem_o[bb])

    fire(0, 0)

    def pair(p, carry):
        fire(2 * p + 1, 1)
        process(2 * p, 0)
        fire(2 * p + 2, 0)
        process(2 * p + 1, 1)
        return carry

    lax.fori_loop(0, TPW // 2, pair, 0)
    process(TPW - 1, 0)
    for t_last in (TPW - 2, TPW - 1):
        bb = t_last % 2

        @pl.when(col(t_last) < FULL_TILES)
        def _():
            pltpu.make_async_copy(
                tout[bb], out.at[pl.ds(col(t_last) * E, E)], sem_o[bb]
            ).wait()

    @pl.when(wid == 4)
    def _():
        nrows = TAIL * E // 128
        pltpu.sync_copy(tail16, tin[0].at[pl.ds(0, nrows)])
        pltpu.sync_copy(
            tin[0].at[pl.ds(0, nrows)], out.at[pl.ds(FULL_TILES * E, nrows)]
        )


@functools.partial(
    pl.kernel,
    mesh=_mesh,
    out_type=jax.ShapeDtypeStruct((L, E // 8, B // 128, 8, 128), jnp.float32),
    scratch_types=[
        pltpu.VMEM((L * 128,), jnp.int32),
        pltpu.VMEM((L, E), jnp.float32),
        [pltpu.VMEM((128,), jnp.int32) for _ in range(2)],
        [pltpu.VMEM((128, E), jnp.float32) for _ in range(2)],
        [pltpu.VMEM((E // 8, 8, 128), jnp.float32) for _ in range(2)],
        [pltpu.SemaphoreType.DMA for _ in range(2)],
        [pltpu.SemaphoreType.DMA for _ in range(2)],
    ],
    compiler_params=pltpu.CompilerParams(
        use_tc_tiling_on_sc=False, needs_layout_passes=False
    ),
)
def _gather_embed(x1d, tab, pos, out, xblk, pos_v, icol, rbuf, obuf, sem_g, sem_w):
    wid = lax.axis_index("s") * 2 + lax.axis_index("c")
    pltpu.sync_copy(x1d.at[pl.ds(wid * 128 * L, 128 * L)], xblk)
    pltpu.sync_copy(pos, pos_v)

    def fire(l, bb):
        @plsc.parallel_loop(0, 8, 1, unroll=8)
        def _(g):
            idx = (_iota16() + 16 * g) * L + l
            icol[bb][pl.ds(16 * g, 16)] = plsc.load_gather(xblk, [idx])

        pltpu.async_copy(tab.at[icol[bb]], rbuf[bb], sem_g[bb])

    def process(l, bb):
        pltpu.make_async_copy(tab.at[icol[bb]], rbuf[bb], sem_g[bb]).wait()

        @pl.when(l >= 2)
        def _():
            for a in range(E // 8):
                pltpu.make_async_copy(
                    obuf[bb].at[a], out.at[l - 2, a, wid], sem_w[bb]
                ).wait()

        l_vec = _iota16() * 0 + l

        @plsc.parallel_loop(0, E, 1, unroll=8)
        def _(e):
            a = e // 8
            b8 = e % 8
            cols = _iota16() * 0 + e
            p_e = plsc.load_gather(pos_v, [l_vec, cols])
            for g in range(8):
                rows = _iota16() + 16 * g
                v = plsc.load_gather(rbuf[bb], [rows, cols])
                obuf[bb][a, b8, pl.ds(16 * g, 16)] = v + p_e

        for a in range(E // 8):
            pltpu.async_copy(obuf[bb].at[a], out.at[l, a, wid], sem_w[bb])

    fire(0, 0)

    def pair(p, carry):
        fire(2 * p + 1, 1)
        process(2 * p, 0)

        @pl.when(p < L // 2 - 1)
        def _():
            fire(2 * p + 2, 0)

        process(2 * p + 1, 1)
        return carry

    lax.fori_loop(0, L // 2, pair, 0)
    for bb in range(2):
        for a in range(E // 8):
            pltpu.make_async_copy(
                obuf[bb].at[a], out.at[L - 2 + bb, a, wid], sem_w[bb]
            ).wait()


def kernel(x, token_emb, pos_emb):
    tail16 = token_emb[FULL_TILES * 128:].reshape(TAIL * E // 128, 128)
    lin = _transpose_table(token_emb.T, tail16)
    tab = lin.reshape(V, E)
    x1d = x.reshape(N).astype(jnp.int32)
    P = _gather_embed(x1d, tab, pos_emb)
    return P.transpose(2, 4, 0, 1, 3).reshape(B, L, E)


# speedup vs baseline: 2.9381x; 1.5077x over previous
import functools

import jax
import jax.numpy as jnp
from jax import lax
from jax.experimental import pallas as pl
from jax.experimental.pallas import tpu as pltpu
from jax.experimental.pallas import tpu_sc as plsc

B = 4096
L = 200
E = 32
V = 1_000_000
N = B * L
NW = 32
FULL_TILES = V // 128
TAIL = V - FULL_TILES * 128
TPW = FULL_TILES // NW + 1

_mesh = plsc.VectorSubcoreMesh(core_axis_name="c", subcore_axis_name="s")


def _iota16():
    return lax.broadcasted_iota(jnp.int32, (16,), 0)


@functools.partial(
    pl.kernel,
    mesh=_mesh,
    out_type=jax.ShapeDtypeStruct((V * E // 128, 128), jnp.float32),
    scratch_types=[
        [pltpu.VMEM((E, 128), jnp.float32) for _ in range(2)],
        [pltpu.VMEM((E, 128), jnp.float32) for _ in range(2)],
        [pltpu.SemaphoreType.DMA for _ in range(2)],
        [pltpu.SemaphoreType.DMA for _ in range(2)],
    ],
    compiler_params=pltpu.CompilerParams(
        use_tc_tiling_on_sc=True, needs_layout_passes=False
    ),
)
def _transpose_table(tokT, tail16, out, tin, tout, sem_i, sem_o):
    wid = lax.axis_index("s") * 2 + lax.axis_index("c")

    def col(t):
        return wid + NW * t

    def fire(t, bb):
        @pl.when(col(t) < FULL_TILES)
        def _():
            pltpu.async_copy(tokT.at[:, pl.ds(col(t) * 128, 128)], tin[bb], sem_i[bb])

    def process(t, bb):
        c = col(t)

        @pl.when(c < FULL_TILES)
        def _():
            pltpu.make_async_copy(
                tokT.at[:, pl.ds(c * 128, 128)], tin[bb], sem_i[bb]
            ).wait()

            def drain_prev():
                pltpu.make_async_copy(
                    tout[bb], out.at[pl.ds((c - 2 * NW) * E, E)], sem_o[bb]
                ).wait()

            if isinstance(t, int):
                if t >= 2:
                    drain_prev()
            else:
                pl.when(t >= 2)(drain_prev)

            @plsc.parallel_loop(0, E, 1, unroll=8)
            def _(r):
                for k in range(8):
                    rows = _iota16() + 16 * (k % 2)
                    cols = _iota16() * 0 + (4 * r + k // 2)
                    tout[bb][r, pl.ds(16 * k, 16)] = plsc.load_gather(
                        tin[bb], [rows, cols]
                    )

            pltpu.async_copy(tout[bb], out.at[pl.ds(c * E, E)], sem_o[bb])

    fire(0, 0)

    def pair(p, carry):
        fire(2 * p + 1, 1)
        process(2 * p, 0)
        fire(2 * p + 2, 0)
        process(2 * p + 1, 1)
        return carry

    lax.fori_loop(0, TPW // 2, pair, 0)
    process(TPW - 1, 0)
    for t_last in (TPW - 2, TPW - 1):
        bb = t_last % 2

        @pl.when(col(t_last) < FULL_TILES)
        def _():
            pltpu.make_async_copy(
                tout[bb], out.at[pl.ds(col(t_last) * E, E)], sem_o[bb]
            ).wait()

    @pl.when(wid == 4)
    def _():
        nrows = TAIL * E // 128
        pltpu.sync_copy(tail16, tin[0].at[pl.ds(0, nrows)])
        pltpu.sync_copy(
            tin[0].at[pl.ds(0, nrows)], out.at[pl.ds(FULL_TILES * E, nrows)]
        )


@functools.partial(
    pl.kernel,
    mesh=_mesh,
    out_type=jax.ShapeDtypeStruct((L, E // 8, B // 128, 8, 128), jnp.float32),
    scratch_types=[
        pltpu.VMEM((L * 128,), jnp.int32),
        pltpu.VMEM((L, E), jnp.float32),
        [pltpu.VMEM((128,), jnp.int32) for _ in range(2)],
        [pltpu.VMEM((128, E), jnp.float32) for _ in range(2)],
        [pltpu.VMEM((E, 129), jnp.float32) for _ in range(2)],
        [pltpu.SemaphoreType.DMA for _ in range(2)],
        [pltpu.SemaphoreType.DMA for _ in range(2)],
    ],
    compiler_params=pltpu.CompilerParams(
        use_tc_tiling_on_sc=False, needs_layout_passes=False
    ),
)
def _gather_embed(x1d, tab, pos, out, xblk, pos_v, icol, rbuf, obuf, sem_g, sem_w):
    wid = lax.axis_index("s") * 2 + lax.axis_index("c")
    pltpu.sync_copy(x1d.at[pl.ds(wid * 128 * L, 128 * L)], xblk)
    pltpu.sync_copy(pos, pos_v)

    def fire(l, bb):
        @plsc.parallel_loop(0, 8, 1, unroll=8)
        def _(g):
            idx = (_iota16() + 16 * g) * L + l
            icol[bb][pl.ds(16 * g, 16)] = plsc.load_gather(xblk, [idx])

        pltpu.async_copy(tab.at[icol[bb]], rbuf[bb], sem_g[bb])

    def process(l, bb):
        pltpu.make_async_copy(tab.at[icol[bb]], rbuf[bb], sem_g[bb]).wait()

        @pl.when(l >= 2)
        def _():
            for a in range(E // 8):
                pltpu.make_async_copy(
                    obuf[bb].at[pl.ds(8 * a, 8), pl.ds(0, 128)],
                    out.at[l - 2, a, wid],
                    sem_w[bb],
                ).wait()

        p0 = pos_v[l, pl.ds(0, 16)]
        p1 = pos_v[l, pl.ds(16, 16)]

        @plsc.parallel_loop(0, 128, 1, unroll=8)
        def _(bd):
            cols = _iota16() * 0 + bd
            rows0 = _iota16()
            v0 = rbuf[bb][bd, pl.ds(0, 16)] + p0
            plsc.store_scatter(obuf[bb], [rows0, cols], v0)
            v1 = rbuf[bb][bd, pl.ds(16, 16)] + p1
            plsc.store_scatter(obuf[bb], [rows0 + 16, cols], v1)

        for a in range(E // 8):
            pltpu.async_copy(
                obuf[bb].at[pl.ds(8 * a, 8), pl.ds(0, 128)],
                out.at[l, a, wid],
                sem_w[bb],
            )

    fire(0, 0)

    def pair(p, carry):
        fire(2 * p + 1, 1)
        process(2 * p, 0)

        @pl.when(p < L // 2 - 1)
        def _():
            fire(2 * p + 2, 0)

        process(2 * p + 1, 1)
        return carry

    lax.fori_loop(0, L // 2, pair, 0)
    for bb in range(2):
        for a in range(E // 8):
            pltpu.make_async_copy(
                obuf[bb].at[pl.ds(8 * a, 8), pl.ds(0, 128)],
                out.at[L - 2 + bb, a, wid],
                sem_w[bb],
            ).wait()


def kernel(x, token_emb, pos_emb):
    tail16 = token_emb[FULL_TILES * 128:].reshape(TAIL * E // 128, 128)
    lin = _transpose_table(token_emb.T, tail16)
    tab = lin.reshape(V, E)
    x1d = x.reshape(N).astype(jnp.int32)
    P = _gather_embed(x1d, tab, pos_emb)
    return P.transpose(2, 4, 0, 1, 3).reshape(B, L, E)


# speedup vs baseline: 5.1229x; 1.7436x over previous
import functools

import jax
import jax.numpy as jnp
from jax import lax
from jax.experimental import pallas as pl
from jax.experimental.pallas import tpu as pltpu
from jax.experimental.pallas import tpu_sc as plsc

B = 4096
L = 200
E = 32
V = 1_000_000
N = B * L
NW = 32
FULL_TILES = V // 128
TAIL = V - FULL_TILES * 128
TPW = FULL_TILES // NW + 1

_mesh = plsc.VectorSubcoreMesh(core_axis_name="c", subcore_axis_name="s")


def _iota16():
    return lax.broadcasted_iota(jnp.int32, (16,), 0)


@functools.partial(
    pl.kernel,
    mesh=_mesh,
    out_type=jax.ShapeDtypeStruct((V * E // 128, 128), jnp.float32),
    scratch_types=[
        [pltpu.VMEM((E, 128), jnp.float32) for _ in range(2)],
        [pltpu.VMEM((E, 128), jnp.float32) for _ in range(2)],
        [pltpu.SemaphoreType.DMA for _ in range(2)],
        [pltpu.SemaphoreType.DMA for _ in range(2)],
    ],
    compiler_params=pltpu.CompilerParams(
        use_tc_tiling_on_sc=True, needs_layout_passes=False
    ),
)
def _transpose_table(tokT, tail16, out, tin, tout, sem_i, sem_o):
    wid = lax.axis_index("s") * 2 + lax.axis_index("c")

    def col(t):
        return wid + NW * t

    def fire(t, bb):
        @pl.when(col(t) < FULL_TILES)
        def _():
            pltpu.async_copy(tokT.at[:, pl.ds(col(t) * 128, 128)], tin[bb], sem_i[bb])

    def process(t, bb):
        c = col(t)

        @pl.when(c < FULL_TILES)
        def _():
            pltpu.make_async_copy(
                tokT.at[:, pl.ds(c * 128, 128)], tin[bb], sem_i[bb]
            ).wait()

            def drain_prev():
                pltpu.make_async_copy(
                    tout[bb], out.at[pl.ds((c - 2 * NW) * E, E)], sem_o[bb]
                ).wait()

            if isinstance(t, int):
                if t >= 2:
                    drain_prev()
            else:
                pl.when(t >= 2)(drain_prev)

            @plsc.parallel_loop(0, 16 * 16, 1, unroll=8)
            def _(i):
                s = i % 16
                blk = i // 16
                e0 = (blk % 2) * 16
                d0 = (blk // 2) * 16
                t = (_iota16() + s) & 15
                in_rows = _iota16() + e0
                in_cols = t + d0
                v = plsc.load_gather(tin[bb], [in_rows, in_cols])
                flat = (t << 5) + (_iota16() + (d0 * E + e0))
                plsc.store_scatter(tout[bb], [flat >> 7, flat & 127], v)

            pltpu.async_copy(tout[bb], out.at[pl.ds(c * E, E)], sem_o[bb])

    fire(0, 0)

    def pair(p, carry):
        fire(2 * p + 1, 1)
        process(2 * p, 0)
        fire(2 * p + 2, 0)
        process(2 * p + 1, 1)
        return carry

    lax.fori_loop(0, TPW // 2, pair, 0)
    process(TPW - 1, 0)
    for t_last in (TPW - 2, TPW - 1):
        bb = t_last % 2

        @pl.when(col(t_last) < FULL_TILES)
        def _():
            pltpu.make_async_copy(
                tout[bb], out.at[pl.ds(col(t_last) * E, E)], sem_o[bb]
            ).wait()

    @pl.when(wid == 4)
    def _():
        nrows = TAIL * E // 128
        pltpu.sync_copy(tail16, tin[0].at[pl.ds(0, nrows)])
        pltpu.sync_copy(
            tin[0].at[pl.ds(0, nrows)], out.at[pl.ds(FULL_TILES * E, nrows)]
        )


@functools.partial(
    pl.kernel,
    mesh=_mesh,
    out_type=jax.ShapeDtypeStruct((L, E // 8, B // 128, 8, 128), jnp.float32),
    scratch_types=[
        pltpu.VMEM((L * 128,), jnp.int32),
        pltpu.VMEM((L, E), jnp.float32),
        [pltpu.VMEM((128,), jnp.int32) for _ in range(2)],
        [pltpu.VMEM((128, E), jnp.float32) for _ in range(2)],
        [pltpu.VMEM((E, 129), jnp.float32) for _ in range(2)],
        [pltpu.SemaphoreType.DMA for _ in range(2)],
        [pltpu.SemaphoreType.DMA for _ in range(2)],
    ],
    compiler_params=pltpu.CompilerParams(
        use_tc_tiling_on_sc=False, needs_layout_passes=False
    ),
)
def _gather_embed(x1d, tab, pos, out, xblk, pos_v, icol, rbuf, obuf, sem_g, sem_w):
    wid = lax.axis_index("s") * 2 + lax.axis_index("c")
    pltpu.sync_copy(x1d.at[pl.ds(wid * 128 * L, 128 * L)], xblk)
    pltpu.sync_copy(pos, pos_v)

    def fire(l, bb):
        @plsc.parallel_loop(0, 8, 1, unroll=8)
        def _(g):
            idx = (_iota16() + 16 * g) * L + l
            icol[bb][pl.ds(16 * g, 16)] = plsc.load_gather(xblk, [idx])

        pltpu.async_copy(tab.at[icol[bb]], rbuf[bb], sem_g[bb])

    def process(l, bb):
        pltpu.make_async_copy(tab.at[icol[bb]], rbuf[bb], sem_g[bb]).wait()

        @pl.when(l >= 2)
        def _():
            for a in range(E // 8):
                pltpu.make_async_copy(
                    obuf[bb].at[pl.ds(8 * a, 8), pl.ds(0, 128)],
                    out.at[l - 2, a, wid],
                    sem_w[bb],
                ).wait()

        p0 = pos_v[l, pl.ds(0, 16)]
        p1 = pos_v[l, pl.ds(16, 16)]

        @plsc.parallel_loop(0, 128, 1, unroll=8)
        def _(bd):
            cols = _iota16() * 0 + bd
            rows0 = _iota16()
            v0 = rbuf[bb][bd, pl.ds(0, 16)] + p0
            plsc.store_scatter(obuf[bb], [rows0, cols], v0)
            v1 = rbuf[bb][bd, pl.ds(16, 16)] + p1
            plsc.store_scatter(obuf[bb], [rows0 + 16, cols], v1)

        for a in range(E // 8):
            pltpu.async_copy(
                obuf[bb].at[pl.ds(8 * a, 8), pl.ds(0, 128)],
                out.at[l, a, wid],
                sem_w[bb],
            )

    fire(0, 0)

    def pair(p, carry):
        fire(2 * p + 1, 1)
        process(2 * p, 0)

        @pl.when(p < L // 2 - 1)
        def _():
            fire(2 * p + 2, 0)

        process(2 * p + 1, 1)
        return carry

    lax.fori_loop(0, L // 2, pair, 0)
    for bb in range(2):
        for a in range(E // 8):
            pltpu.make_async_copy(
                obuf[bb].at[pl.ds(8 * a, 8), pl.ds(0, 128)],
                out.at[L - 2 + bb, a, wid],
                sem_w[bb],
            ).wait()


def kernel(x, token_emb, pos_emb):
    tail16 = token_emb[FULL_TILES * 128:].reshape(TAIL * E // 128, 128)
    lin = _transpose_table(token_emb.T, tail16)
    tab = lin.reshape(V, E)
    x1d = x.reshape(N).astype(jnp.int32)
    P = _gather_embed(x1d, tab, pos_emb)
    return P.transpose(2, 4, 0, 1, 3).reshape(B, L, E)


# speedup vs baseline: 5.2406x; 1.0230x over previous
import functools

import jax
import jax.numpy as jnp
from jax import lax
from jax.experimental import pallas as pl
from jax.experimental.pallas import tpu as pltpu
from jax.experimental.pallas import tpu_sc as plsc

B = 4096
L = 200
E = 32
V = 1_000_000
N = B * L
NW = 32
FULL_TILES = V // 128
TAIL = V - FULL_TILES * 128
TPW = FULL_TILES // NW + 1

_mesh = plsc.VectorSubcoreMesh(core_axis_name="c", subcore_axis_name="s")


def _iota16():
    return lax.broadcasted_iota(jnp.int32, (16,), 0)


@functools.partial(
    pl.kernel,
    mesh=_mesh,
    out_type=jax.ShapeDtypeStruct((V * E // 128, 128), jnp.float32),
    scratch_types=[
        [pltpu.VMEM((E, 128), jnp.float32) for _ in range(2)],
        [pltpu.VMEM((E, 128), jnp.float32) for _ in range(2)],
        [pltpu.SemaphoreType.DMA for _ in range(2)],
        [pltpu.SemaphoreType.DMA for _ in range(2)],
    ],
    compiler_params=pltpu.CompilerParams(
        use_tc_tiling_on_sc=True, needs_layout_passes=False
    ),
)
def _transpose_table(tokT, tail16, out, tin, tout, sem_i, sem_o):
    wid = lax.axis_index("s") * 2 + lax.axis_index("c")

    def col(t):
        return wid + NW * t

    def fire(t, bb):
        @pl.when(col(t) < FULL_TILES)
        def _():
            pltpu.async_copy(tokT.at[:, pl.ds(col(t) * 128, 128)], tin[bb], sem_i[bb])

    def process(t, bb):
        c = col(t)

        @pl.when(c < FULL_TILES)
        def _():
            pltpu.make_async_copy(
                tokT.at[:, pl.ds(c * 128, 128)], tin[bb], sem_i[bb]
            ).wait()

            def drain_prev():
                pltpu.make_async_copy(
                    tout[bb], out.at[pl.ds((c - 2 * NW) * E, E)], sem_o[bb]
                ).wait()

            if isinstance(t, int):
                if t >= 2:
                    drain_prev()
            else:
                pl.when(t >= 2)(drain_prev)

            iota = _iota16()
            iota32 = iota << 5

            @plsc.parallel_loop(0, 16, 1, unroll=2)
            def _(s):
                t = (iota + s) & 15
                for e0 in (0, 16):
                    in_rows = t + e0
                    for db in range(8):
                        d0 = 16 * db
                        v = plsc.load_gather(tin[bb], [in_rows, iota + d0])
                        flat = (iota32 + (d0 * E + e0)) + t
                        plsc.store_scatter(tout[bb], [flat >> 7, flat & 127], v)

            pltpu.async_copy(tout[bb], out.at[pl.ds(c * E, E)], sem_o[bb])

    fire(0, 0)

    def pair(p, carry):
        fire(2 * p + 1, 1)
        process(2 * p, 0)
        fire(2 * p + 2, 0)
        process(2 * p + 1, 1)
        return carry

    lax.fori_loop(0, TPW // 2, pair, 0)
    process(TPW - 1, 0)
    for t_last in (TPW - 2, TPW - 1):
        bb = t_last % 2

        @pl.when(col(t_last) < FULL_TILES)
        def _():
            pltpu.make_async_copy(
                tout[bb], out.at[pl.ds(col(t_last) * E, E)], sem_o[bb]
            ).wait()

    @pl.when(wid == 4)
    def _():
        nrows = TAIL * E // 128
        pltpu.sync_copy(tail16, tin[0].at[pl.ds(0, nrows)])
        pltpu.sync_copy(
            tin[0].at[pl.ds(0, nrows)], out.at[pl.ds(FULL_TILES * E, nrows)]
        )


@functools.partial(
    pl.kernel,
    mesh=_mesh,
    out_type=jax.ShapeDtypeStruct((L, E // 8, B // 128, 8, 128), jnp.float32),
    scratch_types=[
        pltpu.VMEM((L // 8, 8, 128), jnp.int32),
        pltpu.VMEM((L, E), jnp.float32),
        [pltpu.VMEM((128, E), jnp.float32) for _ in range(2)],
        [pltpu.VMEM((E, 129), jnp.float32) for _ in range(2)],
        pltpu.SemaphoreType.DMA,
        [pltpu.SemaphoreType.DMA for _ in range(2)],
        [pltpu.SemaphoreType.DMA for _ in range(2)],
    ],
    compiler_params=pltpu.CompilerParams(
        use_tc_tiling_on_sc=False, needs_layout_passes=False
    ),
)
def _gather_embed(xP, tab, pos, out, xblk, pos_v, rbuf, obuf, sem_x, sem_g, sem_w):
    wid = lax.axis_index("s") * 2 + lax.axis_index("c")
    for g in range(L // 8):
        pltpu.async_copy(xP.at[g, wid], xblk.at[g], sem_x)
    pltpu.sync_copy(pos, pos_v)
    for g in range(L // 8):
        pltpu.make_async_copy(xP.at[g, wid], xblk.at[g], sem_x).wait()

    def fire(l, bb):
        pltpu.async_copy(tab.at[xblk.at[l // 8, l % 8]], rbuf[bb], sem_g[bb])

    def process(l, bb):
        pltpu.make_async_copy(
            tab.at[xblk.at[l // 8, l % 8]], rbuf[bb], sem_g[bb]
        ).wait()

        @pl.when(l >= 2)
        def _():
            for a in range(E // 8):
                pltpu.make_async_copy(
                    obuf[bb].at[pl.ds(8 * a, 8), pl.ds(0, 128)],
                    out.at[l - 2, a, wid],
                    sem_w[bb],
                ).wait()

        p0 = pos_v[l, pl.ds(0, 16)]
        p1 = pos_v[l, pl.ds(16, 16)]

        @plsc.parallel_loop(0, 128, 1, unroll=8)
        def _(bd):
            cols = _iota16() * 0 + bd
            rows0 = _iota16()
            v0 = rbuf[bb][bd, pl.ds(0, 16)] + p0
            plsc.store_scatter(obuf[bb], [rows0, cols], v0)
            v1 = rbuf[bb][bd, pl.ds(16, 16)] + p1
            plsc.store_scatter(obuf[bb], [rows0 + 16, cols], v1)

        for a in range(E // 8):
            pltpu.async_copy(
                obuf[bb].at[pl.ds(8 * a, 8), pl.ds(0, 128)],
                out.at[l, a, wid],
                sem_w[bb],
            )

    fire(0, 0)

    def pair(p, carry):
        fire(2 * p + 1, 1)
        process(2 * p, 0)

        @pl.when(p < L // 2 - 1)
        def _():
            fire(2 * p + 2, 0)

        process(2 * p + 1, 1)
        return carry

    lax.fori_loop(0, L // 2, pair, 0)
    for bb in range(2):
        for a in range(E // 8):
            pltpu.make_async_copy(
                obuf[bb].at[pl.ds(8 * a, 8), pl.ds(0, 128)],
                out.at[L - 2 + bb, a, wid],
                sem_w[bb],
            ).wait()


def kernel(x, token_emb, pos_emb):
    tail16 = token_emb[FULL_TILES * 128:].reshape(TAIL * E // 128, 128)
    lin = _transpose_table(token_emb.T, tail16)
    tab = lin.reshape(V, E)
    xP = (
        x.astype(jnp.int32)
        .T.reshape(L // 8, 8, B // 128, 128)
        .transpose(0, 2, 1, 3)
    )
    P = _gather_embed(xP, tab, pos_emb)
    return P.transpose(2, 4, 0, 1, 3).reshape(B, L, E)
